# initial kernel scaffold (unmeasured)
import jax
import jax.numpy as jnp
from jax import lax
from jax.experimental import pallas as pl
from jax.experimental.pallas import tpu as pltpu

N_DEV = 4
B, Sq, Skv, Hq, Dh = 2, 512, 2048, 8, 64
S_loc = Skv // N_DEV
BH = B * Hq
D_model = 768
D_qk = Hq * Dh
BLK = 64


def kernel(x, Wq, K_ext, V_ext, Wo):
    bf16 = jnp.bfloat16
    x16 = x.astype(bf16)
    wq16 = Wq.astype(bf16)
    wo16 = Wo.astype(bf16)
    k_t = jnp.transpose(K_ext.astype(bf16), (0, 2, 1, 3)).reshape(BH, S_loc, Dh)
    v_t = jnp.transpose(V_ext.astype(bf16), (0, 2, 1, 3)).reshape(BH, S_loc, Dh)
    kv = jnp.stack([k_t, v_t])

    def body(x_ref, wq_ref, kv_ref, wo_ref, out_ref,
             kv_full, comm, q_scr, ctx, send_sems, recv_sems):
        my = lax.axis_index("i")
        left = lax.rem(my + N_DEV - 1, N_DEV)
        right = lax.rem(my + 1, N_DEV)

        barrier_sem = pltpu.get_barrier_semaphore()
        for nbr in (left, right):
            pl.semaphore_signal(
                barrier_sem, inc=1,
                device_id=(nbr,), device_id_type=pl.DeviceIdType.MESH,
            )
        pl.semaphore_wait(barrier_sem, 2)

        kv_full[:, :, pl.ds(my * S_loc, S_loc), :] = kv_ref[...]
        comm[0] = kv_ref[...]

        for h in range(N_DEV - 1):
            send_slot = h % 2
            recv_slot = (h + 1) % 2
            rdma = pltpu.make_async_remote_copy(
                src_ref=comm.at[send_slot],
                dst_ref=comm.at[recv_slot],
                send_sem=send_sems.at[send_slot],
                recv_sem=recv_sems.at[recv_slot],
                device_id=(right,),
                device_id_type=pl.DeviceIdType.MESH,
            )
            rdma.start()
            rdma.wait()
            origin = lax.rem(my + N_DEV - h - 1, N_DEV)
            kv_full[:, :, pl.ds(origin * S_loc, S_loc), :] = comm[recv_slot]

        for b in range(B):
            q_scr[b] = jnp.dot(
                x_ref[b], wq_ref[...], preferred_element_type=jnp.float32
            ).astype(bf16)

        qb = lax.broadcasted_iota(jnp.int32, (Sq, Skv), 0) // BLK
        kb = lax.broadcasted_iota(jnp.int32, (Sq, Skv), 1) // BLK
        mask = (qb == kb) | (kb == 0) | (lax.rem(qb + kb, 3) == 0)

        for b in range(B):
            for h in range(Hq):
                q = q_scr[b, :, h * Dh:(h + 1) * Dh]
                k = kv_full[0, b * Hq + h]
                s = lax.dot_general(
                    q, k, (((1,), (1,)), ((), ())),
                    preferred_element_type=jnp.float32,
                ) * 0.125
                s = jnp.where(mask, s, -1e9)
                m = jnp.max(s, axis=1, keepdims=True)
                w = jnp.exp(s - m)
                w = w / jnp.sum(w, axis=1, keepdims=True)
                v = kv_full[1, b * Hq + h]
                c = jnp.dot(
                    w.astype(bf16), v, preferred_element_type=jnp.float32
                )
                ctx[b, :, h * Dh:(h + 1) * Dh] = c.astype(bf16)

        for b in range(B):
            out_ref[b] = jnp.dot(
                ctx[b], wo_ref[...], preferred_element_type=jnp.float32
            )

    return pl.pallas_call(
        body,
        out_shape=jax.ShapeDtypeStruct((B, Sq, D_model), jnp.float32),
        in_specs=[
            pl.BlockSpec(memory_space=pltpu.VMEM),
            pl.BlockSpec(memory_space=pltpu.VMEM),
            pl.BlockSpec(memory_space=pltpu.VMEM),
            pl.BlockSpec(memory_space=pltpu.VMEM),
        ],
        out_specs=pl.BlockSpec(memory_space=pltpu.VMEM),
        scratch_shapes=[
            pltpu.VMEM((2, BH, Skv, Dh), bf16),
            pltpu.VMEM((2, 2, BH, S_loc, Dh), bf16),
            pltpu.VMEM((B, Sq, D_qk), bf16),
            pltpu.VMEM((B, Sq, D_qk), bf16),
            pltpu.SemaphoreType.DMA((2,)),
            pltpu.SemaphoreType.DMA((2,)),
        ],
        compiler_params=pltpu.CompilerParams(collective_id=0),
    )(x16, wq16, kv, wo16)


# baseline (device time: 205232 ns/iter reference)
import jax
import jax.numpy as jnp
from jax import lax
from jax.experimental import pallas as pl
from jax.experimental.pallas import tpu as pltpu

N_DEV = 4
B, Sq, Skv, Hq, Dh = 2, 512, 2048, 8, 64
S_loc = Skv // N_DEV
BH = B * Hq
D_model = 768
D_qk = Hq * Dh
BLK = 64


def kernel(x, Wq, K_ext, V_ext, Wo):
    bf16 = jnp.bfloat16
    x16 = x.astype(bf16)
    wq16 = Wq.astype(bf16)
    wo16 = Wo.astype(bf16)
    k_t = jnp.transpose(K_ext.astype(bf16), (0, 2, 1, 3)).reshape(BH, S_loc, Dh)
    v_t = jnp.transpose(V_ext.astype(bf16), (0, 2, 1, 3)).reshape(BH, S_loc, Dh)
    kv = jnp.stack([k_t, v_t])

    def body(x_ref, wq_ref, kv_ref, wo_ref, out_ref,
             kv_full, comm, q_scr, ctx, send_sems, recv_sems):
        my = lax.axis_index("i")
        left = lax.rem(my + N_DEV - 1, N_DEV)
        right = lax.rem(my + 1, N_DEV)

        barrier_sem = pltpu.get_barrier_semaphore()
        for nbr in (left, right):
            pl.semaphore_signal(
                barrier_sem, inc=1,
                device_id=(nbr,), device_id_type=pl.DeviceIdType.MESH,
            )
        pl.semaphore_wait(barrier_sem, 2)

        kv_full[:, :, pl.ds(my * S_loc, S_loc), :] = kv_ref[...]
        comm[0] = kv_ref[...]

        for h in range(N_DEV - 1):
            send_slot = h % 2
            recv_slot = (h + 1) % 2
            rdma = pltpu.make_async_remote_copy(
                src_ref=comm.at[send_slot],
                dst_ref=comm.at[recv_slot],
                send_sem=send_sems.at[send_slot],
                recv_sem=recv_sems.at[recv_slot],
                device_id=(right,),
                device_id_type=pl.DeviceIdType.MESH,
            )
            rdma.start()
            rdma.wait()
            origin = lax.rem(my + N_DEV - h - 1, N_DEV)
            kv_full[:, :, pl.ds(origin * S_loc, S_loc), :] = comm[recv_slot]

        for b in range(B):
            q_scr[b] = jnp.dot(
                x_ref[b], wq_ref[...], preferred_element_type=jnp.float32
            ).astype(bf16)

        QC = 256
        for qc in range(Sq // QC):
            qb = (lax.broadcasted_iota(jnp.int32, (QC, Skv), 0)
                  + qc * QC) // BLK
            kb = lax.broadcasted_iota(jnp.int32, (QC, Skv), 1) // BLK
            mask = (qb == kb) | (kb == 0) | (lax.rem(qb + kb, 3) == 0)
            bias = jnp.where(mask, 0.0, -1e9)
            for b in range(B):
                for h in range(Hq):
                    q = q_scr[b, qc * QC:(qc + 1) * QC,
                              h * Dh:(h + 1) * Dh]
                    k = kv_full[0, b * Hq + h]
                    s = lax.dot_general(
                        q, k, (((1,), (1,)), ((), ())),
                        preferred_element_type=jnp.float32,
                    ) * 0.125 + bias
                    m = jnp.max(s, axis=1, keepdims=True)
                    w = jnp.exp(s - m)
                    w = w / jnp.sum(w, axis=1, keepdims=True)
                    v = kv_full[1, b * Hq + h]
                    c = jnp.dot(
                        w.astype(bf16), v, preferred_element_type=jnp.float32
                    )
                    ctx[b, qc * QC:(qc + 1) * QC,
                        h * Dh:(h + 1) * Dh] = c.astype(bf16)

        for b in range(B):
            out_ref[b] = jnp.dot(
                ctx[b], wo_ref[...], preferred_element_type=jnp.float32
            )

    return pl.pallas_call(
        body,
        out_shape=jax.ShapeDtypeStruct((B, Sq, D_model), jnp.float32),
        in_specs=[
            pl.BlockSpec(memory_space=pltpu.VMEM),
            pl.BlockSpec(memory_space=pltpu.VMEM),
            pl.BlockSpec(memory_space=pltpu.VMEM),
            pl.BlockSpec(memory_space=pltpu.VMEM),
        ],
        out_specs=pl.BlockSpec(memory_space=pltpu.VMEM),
        scratch_shapes=[
            pltpu.VMEM((2, BH, Skv, Dh), bf16),
            pltpu.VMEM((2, 2, BH, S_loc, Dh), bf16),
            pltpu.VMEM((B, Sq, D_qk), bf16),
            pltpu.VMEM((B, Sq, D_qk), bf16),
            pltpu.SemaphoreType.DMA((2,)),
            pltpu.SemaphoreType.DMA((2,)),
        ],
        compiler_params=pltpu.CompilerParams(
            collective_id=0, vmem_limit_bytes=64 * 1024 * 1024
        ),
    )(x16, wq16, kv, wo16)


# device time: 84886 ns/iter; 2.4177x vs baseline; 2.4177x over previous
import jax
import jax.numpy as jnp
from jax import lax
from jax.experimental import pallas as pl
from jax.experimental.pallas import tpu as pltpu

N_DEV = 4
B, Sq, Skv, Hq, Dh = 2, 512, 2048, 8, 64
S_loc = Skv // N_DEV
BH = B * Hq
D_model = 768
D_qk = Hq * Dh
BLK = 64


def kernel(x, Wq, K_ext, V_ext, Wo):
    bf16 = jnp.bfloat16
    x16 = x.astype(bf16)
    wq16 = Wq.astype(bf16)
    wo16 = Wo.astype(bf16)
    k_t = jnp.transpose(K_ext.astype(bf16), (0, 2, 1, 3)).reshape(BH, S_loc, Dh)
    v_t = jnp.transpose(V_ext.astype(bf16), (0, 2, 1, 3)).reshape(BH, S_loc, Dh)

    def body(x_ref, wq_ref, k_ref, v_ref, wo_ref, out_ref,
             q_scr, send_ctx, recv_ctx, send_l, recv_l, ctx_comb,
             send_sems, recv_sems):
        my = lax.axis_index("i")
        p1 = my + 1 - 2 * lax.rem(my, 2)
        p2 = 3 - my

        barrier_sem = pltpu.get_barrier_semaphore()
        for nbr in (p1, p2):
            pl.semaphore_signal(
                barrier_sem, inc=1,
                device_id=(nbr,), device_id_type=pl.DeviceIdType.MESH,
            )
        pl.semaphore_wait(barrier_sem, 2)

        for b in range(B):
            q_scr[b] = jnp.dot(
                x_ref[b], wq_ref[...], preferred_element_type=jnp.float32
            ).astype(jnp.bfloat16)

        qb = lax.broadcasted_iota(jnp.int32, (Sq, S_loc), 0) // BLK
        kb = lax.broadcasted_iota(jnp.int32, (Sq, S_loc), 1) // BLK + my * 8
        mask = (qb == kb) | (kb == 0) | (lax.rem(qb + kb, 3) == 0)
        bias = jnp.where(mask, 0.0, -1e9).astype(jnp.float32)

        for b in range(B):
            for h in range(Hq):
                bh = b * Hq + h
                q = q_scr[b, :, h * Dh:(h + 1) * Dh]
                s = lax.dot_general(
                    q, k_ref[bh], (((1,), (1,)), ((), ())),
                    preferred_element_type=jnp.float32,
                ) * 0.125 + bias
                w = jnp.exp(s)
                send_l[:, bh:bh + 1] = jnp.sum(w, axis=1, keepdims=True)
                c = jnp.dot(
                    w.astype(jnp.bfloat16), v_ref[bh],
                    preferred_element_type=jnp.float32,
                )
                send_ctx[bh] = c.astype(jnp.bfloat16)

        for step, partner in enumerate((p1, p2)):
            rdma_c = pltpu.make_async_remote_copy(
                src_ref=send_ctx,
                dst_ref=recv_ctx.at[step],
                send_sem=send_sems.at[step, 0],
                recv_sem=recv_sems.at[step, 0],
                device_id=(partner,),
                device_id_type=pl.DeviceIdType.MESH,
            )
            rdma_l = pltpu.make_async_remote_copy(
                src_ref=send_l,
                dst_ref=recv_l.at[step],
                send_sem=send_sems.at[step, 1],
                recv_sem=recv_sems.at[step, 1],
                device_id=(partner,),
                device_id_type=pl.DeviceIdType.MESH,
            )
            rdma_c.start()
            rdma_l.start()
            rdma_c.wait()
            rdma_l.wait()
            if step == 0:
                send_ctx[...] = (
                    send_ctx[...].astype(jnp.float32)
                    + recv_ctx[0].astype(jnp.float32)
                ).astype(jnp.bfloat16)
                send_l[...] = send_l[...] + recv_l[0]

        l_g = send_l[...] + recv_l[1]
        for b in range(B):
            for h in range(Hq):
                bh = b * Hq + h
                c = (
                    send_ctx[bh].astype(jnp.float32)
                    + recv_ctx[1, bh].astype(jnp.float32)
                ) / l_g[:, bh:bh + 1]
                ctx_comb[b, :, h * Dh:(h + 1) * Dh] = c.astype(jnp.bfloat16)

        for b in range(B):
            out_ref[b] = jnp.dot(
                ctx_comb[b], wo_ref[...], preferred_element_type=jnp.float32
            )

    return pl.pallas_call(
        body,
        out_shape=jax.ShapeDtypeStruct((B, Sq, D_model), jnp.float32),
        in_specs=[pl.BlockSpec(memory_space=pltpu.VMEM)] * 5,
        out_specs=pl.BlockSpec(memory_space=pltpu.VMEM),
        scratch_shapes=[
            pltpu.VMEM((B, Sq, D_qk), jnp.bfloat16),
            pltpu.VMEM((BH, Sq, Dh), jnp.bfloat16),
            pltpu.VMEM((2, BH, Sq, Dh), jnp.bfloat16),
            pltpu.VMEM((Sq, BH), jnp.float32),
            pltpu.VMEM((2, Sq, BH), jnp.float32),
            pltpu.VMEM((B, Sq, D_qk), jnp.bfloat16),
            pltpu.SemaphoreType.DMA((2, 2)),
            pltpu.SemaphoreType.DMA((2, 2)),
        ],
        compiler_params=pltpu.CompilerParams(
            collective_id=0, vmem_limit_bytes=64 * 1024 * 1024
        ),
    )(x16, wq16, k_t, v_t, wo16)


# device time: 73506 ns/iter; 2.7920x vs baseline; 1.1548x over previous
import jax
import jax.numpy as jnp
from jax import lax
from jax.experimental import pallas as pl
from jax.experimental.pallas import tpu as pltpu

N_DEV = 4
B, Sq, Skv, Hq, Dh = 2, 512, 2048, 8, 64
S_loc = Skv // N_DEV
BH = B * Hq
D_model = 768
D_qk = Hq * Dh
BLK = 64


def kernel(x, Wq, K_ext, V_ext, Wo):
    bf16 = jnp.bfloat16
    x16 = x.astype(bf16)
    wq16 = (Wq * 0.125).astype(bf16)
    wo16 = Wo.astype(bf16)
    k_t = jnp.transpose(K_ext.astype(bf16), (0, 2, 1, 3)).reshape(BH, S_loc, Dh)
    v_t = jnp.transpose(V_ext.astype(bf16), (0, 2, 1, 3)).reshape(BH, S_loc, Dh)
    v_pad = jnp.concatenate(
        [
            v_t,
            jnp.ones((BH, S_loc, 1), bf16),
            jnp.zeros((BH, S_loc, Dh - 1), bf16),
        ],
        axis=-1,
    )

    def body(x_ref, wq_ref, k_ref, v_ref, wo_ref, out_ref,
             q_scr, send_ctx, recv_ctx, send_l, recv_l, ctx_comb,
             send_sems, recv_sems):
        my = lax.axis_index("i")
        p1 = my + 1 - 2 * lax.rem(my, 2)
        p2 = 3 - my

        barrier_sem = pltpu.get_barrier_semaphore()
        for nbr in (p1, p2):
            pl.semaphore_signal(
                barrier_sem, inc=1,
                device_id=(nbr,), device_id_type=pl.DeviceIdType.MESH,
            )
        pl.semaphore_wait(barrier_sem, 2)

        qb = lax.broadcasted_iota(jnp.int32, (Sq, S_loc), 0) // BLK
        kb = lax.broadcasted_iota(jnp.int32, (Sq, S_loc), 1) // BLK + my * 8
        mask = (qb == kb) | (kb == 0) | (lax.rem(qb + kb, 3) == 0)
        bias = jnp.where(mask, 0.0, -1e9).astype(jnp.float32)

        def make_rdmas(step, g, partner):
            rdma_c = pltpu.make_async_remote_copy(
                src_ref=send_ctx.at[g],
                dst_ref=recv_ctx.at[step, g],
                send_sem=send_sems.at[step, g, 0],
                recv_sem=recv_sems.at[step, g, 0],
                device_id=(partner,),
                device_id_type=pl.DeviceIdType.MESH,
            )
            rdma_l = pltpu.make_async_remote_copy(
                src_ref=send_l.at[g],
                dst_ref=recv_l.at[step, g],
                send_sem=send_sems.at[step, g, 1],
                recv_sem=recv_sems.at[step, g, 1],
                device_id=(partner,),
                device_id_type=pl.DeviceIdType.MESH,
            )
            return rdma_c, rdma_l

        ex1 = []
        for g in range(B):
            q_scr[g] = jnp.dot(
                x_ref[g], wq_ref[...], preferred_element_type=jnp.float32
            ).astype(jnp.bfloat16)
            for h in range(Hq):
                bh = g * Hq + h
                q = q_scr[g, :, h * Dh:(h + 1) * Dh]
                s = lax.dot_general(
                    q, k_ref[bh], (((1,), (1,)), ((), ())),
                    preferred_element_type=jnp.float32,
                ) + bias
                w = jnp.exp(s)
                cl = jnp.dot(
                    w.astype(jnp.bfloat16), v_ref[bh],
                    preferred_element_type=jnp.float32,
                )
                send_ctx[g, h] = cl[:, :Dh].astype(jnp.bfloat16)
                send_l[g, :, h:h + 1] = cl[:, Dh:Dh + 1]
            rc, rl = make_rdmas(0, g, p1)
            rc.start()
            rl.start()
            ex1.append((rc, rl))

        ex2 = []
        for g in range(B):
            rc, rl = ex1[g]
            rc.wait()
            rl.wait()
            send_ctx[g] = (
                send_ctx[g].astype(jnp.float32)
                + recv_ctx[0, g].astype(jnp.float32)
            ).astype(jnp.bfloat16)
            send_l[g] = send_l[g] + recv_l[0, g]
            rc2, rl2 = make_rdmas(1, g, p2)
            rc2.start()
            rl2.start()
            ex2.append((rc2, rl2))

        for g in range(B):
            rc2, rl2 = ex2[g]
            rc2.wait()
            rl2.wait()
            l_g = send_l[g] + recv_l[1, g]
            for h in range(Hq):
                c = (
                    send_ctx[g, h].astype(jnp.float32)
                    + recv_ctx[1, g, h].astype(jnp.float32)
                ) / l_g[:, h:h + 1]
                ctx_comb[g, :, h * Dh:(h + 1) * Dh] = c.astype(jnp.bfloat16)
            out_ref[g] = jnp.dot(
                ctx_comb[g], wo_ref[...], preferred_element_type=jnp.float32
            )

    return pl.pallas_call(
        body,
        out_shape=jax.ShapeDtypeStruct((B, Sq, D_model), jnp.float32),
        in_specs=[pl.BlockSpec(memory_space=pltpu.VMEM)] * 5,
        out_specs=pl.BlockSpec(memory_space=pltpu.VMEM),
        scratch_shapes=[
            pltpu.VMEM((B, Sq, D_qk), jnp.bfloat16),
            pltpu.VMEM((B, Hq, Sq, Dh), jnp.bfloat16),
            pltpu.VMEM((2, B, Hq, Sq, Dh), jnp.bfloat16),
            pltpu.VMEM((B, Sq, Hq), jnp.float32),
            pltpu.VMEM((2, B, Sq, Hq), jnp.float32),
            pltpu.VMEM((B, Sq, D_qk), jnp.bfloat16),
            pltpu.SemaphoreType.DMA((2, B, 2)),
            pltpu.SemaphoreType.DMA((2, B, 2)),
        ],
        compiler_params=pltpu.CompilerParams(
            collective_id=0, vmem_limit_bytes=64 * 1024 * 1024
        ),
    )(x16, wq16, k_t, v_pad, wo16)
